# Initial kernel scaffold; baseline (speedup 1.0000x reference)
import jax
import jax.numpy as jnp
from jax.experimental import pallas as pl


def kernel(x, edge_index, W1, b1, W2, b2):
    raise NotImplementedError("write your pallas kernel here")


# speedup vs baseline: 8.1596x; 8.1596x over previous
import functools

import jax
import jax.numpy as jnp
from jax import lax
from jax.experimental import pallas as pl
from jax.experimental.pallas import tpu as pltpu
from jax.experimental.pallas import tpu_sc as plsc

N = 10000
D = 128
E = 320000

NC = 2
NS = 16
NW = NC * NS
NPAD = 10240
RPS = NPAD // NS
CHUNK = 128
CH = 80
EW = CH * CHUNK
EPAD = NW * EW

BM = 1024


def _sc_mesh():
    return plsc.VectorSubcoreMesh(core_axis_name="c", subcore_axis_name="s")


@functools.partial(
    pl.kernel,
    out_type=jax.ShapeDtypeStruct((NC, NPAD, D), jnp.float32),
    mesh=_sc_mesh(),
    scratch_types=[
        pltpu.VMEM((CH, CHUNK), jnp.int32),
        pltpu.VMEM((CHUNK, D), jnp.float32),
        pltpu.VMEM((CHUNK, D), jnp.float32),
        pltpu.VMEM((RPS // CHUNK, CHUNK), jnp.int32),
        pltpu.VMEM_SHARED((NPAD, D), jnp.float32),
    ],
)
def _deg_kernel(dst_hbm, out_hbm, dst_v, ones_v, bounce_v, iota_v, acc_sh):
    cid = lax.axis_index("c")
    sid = lax.axis_index("s")
    wid = sid * NC + cid
    pltpu.sync_copy(dst_hbm.at[wid], dst_v)

    def fill(r, c):
        for m in range(D // 16):
            ones_v[r, pl.ds(m * 16, 16)] = jnp.ones((16,), jnp.float32)
            bounce_v[r, pl.ds(m * 16, 16)] = jnp.zeros((16,), jnp.float32)
        return c
    lax.fori_loop(0, CHUNK, fill, 0)
    base = sid * RPS
    for k in range(RPS // CHUNK):
        for m in range(CHUNK // 16):
            iota_v[k, pl.ds(m * 16, 16)] = (
                lax.iota(jnp.int32, 16) + (base + k * CHUNK + m * 16))

    for k in range(RPS // CHUNK):
        pltpu.sync_copy(bounce_v, acc_sh.at[iota_v.at[k]])
    plsc.subcore_barrier()

    def body(j, c):
        pltpu.sync_copy(ones_v, acc_sh.at[dst_v.at[j]], add=True)
        return c
    lax.fori_loop(0, CH, body, 0)
    plsc.subcore_barrier()

    for k in range(RPS // CHUNK):
        pltpu.sync_copy(acc_sh.at[iota_v.at[k]], bounce_v)
        pltpu.sync_copy(bounce_v,
                        out_hbm.at[cid, pl.ds(sid * RPS + k * CHUNK, CHUNK)])


@functools.partial(
    pl.kernel,
    out_type=jax.ShapeDtypeStruct((NC, NPAD, D), jnp.float32),
    mesh=_sc_mesh(),
    scratch_types=[
        pltpu.VMEM((CH, CHUNK), jnp.int32),
        pltpu.VMEM((CH, CHUNK), jnp.int32),
        pltpu.VMEM((CHUNK, D), jnp.float32),
        pltpu.VMEM((RPS // CHUNK, CHUNK), jnp.int32),
        pltpu.VMEM_SHARED((NPAD, D), jnp.float32),
        pltpu.SemaphoreType.DMA,
    ],
)
def _scatter_kernel(g_hbm, src_hbm, dst_hbm, out_hbm,
                    src_v, dst_v, buf_a, iota_v, acc_sh, sem_a):
    cid = lax.axis_index("c")
    sid = lax.axis_index("s")
    wid = sid * NC + cid
    pltpu.sync_copy(src_hbm.at[wid], src_v)
    pltpu.sync_copy(dst_hbm.at[wid], dst_v)

    def zrow(r, c):
        for k in range(D // 16):
            buf_a[r, pl.ds(k * 16, 16)] = jnp.zeros((16,), jnp.float32)
        return c
    lax.fori_loop(0, CHUNK, zrow, 0)
    base = sid * RPS
    for k in range(RPS // CHUNK):
        for m in range(CHUNK // 16):
            iota_v[k, pl.ds(m * 16, 16)] = (
                lax.iota(jnp.int32, 16) + (base + k * CHUNK + m * 16))
    for k in range(RPS // CHUNK):
        pltpu.sync_copy(buf_a, acc_sh.at[iota_v.at[k]])
    plsc.subcore_barrier()

    def body(j, c):
        pltpu.async_copy(g_hbm.at[src_v.at[j]], buf_a, sem_a).wait()
        pltpu.sync_copy(buf_a, acc_sh.at[dst_v.at[j]], add=True)
        return c
    lax.fori_loop(0, CH, body, 0)

    plsc.subcore_barrier()
    for k in range(RPS // CHUNK):
        pltpu.sync_copy(acc_sh.at[iota_v.at[k]], buf_a)
        pltpu.sync_copy(buf_a, out_hbm.at[cid, pl.ds(sid * RPS + k * CHUNK, CHUNK)])


def _mm_body(x_ref, w_ref, o_ref):
    o_ref[...] = jnp.dot(x_ref[...], w_ref[...],
                         preferred_element_type=jnp.float32)


def _mm(x, w):
    return pl.pallas_call(
        _mm_body,
        grid=(NPAD // BM,),
        in_specs=[pl.BlockSpec((BM, D), lambda i: (i, 0)),
                  pl.BlockSpec((D, D), lambda i: (0, 0))],
        out_specs=pl.BlockSpec((BM, D), lambda i: (i, 0)),
        out_shape=jax.ShapeDtypeStruct((NPAD, D), jnp.float32),
    )(x, w)


def _dinv_of(cnt_blk):
    deg = cnt_blk[0][:, :1] + cnt_blk[1][:, :1] + 1.0
    return lax.rsqrt(deg)


def _fin1_body(cnt_ref, h_ref, g_ref):
    g_ref[...] = h_ref[...] * _dinv_of(cnt_ref[...])


def _fin1(counts, hraw):
    return pl.pallas_call(
        _fin1_body,
        grid=(NPAD // BM,),
        in_specs=[pl.BlockSpec((NC, BM, D), lambda i: (0, i, 0)),
                  pl.BlockSpec((BM, D), lambda i: (i, 0))],
        out_specs=pl.BlockSpec((BM, D), lambda i: (i, 0)),
        out_shape=jax.ShapeDtypeStruct((NPAD, D), jnp.float32),
    )(counts, hraw)


def _mid_body(cnt_ref, acc_ref, g1_ref, b_ref, w_ref, g2_ref):
    dinv = _dinv_of(cnt_ref[...])
    h = (acc_ref[0] + acc_ref[1] + g1_ref[...]) * dinv + b_ref[...]
    h = jnp.maximum(h, 0.0)
    g2_ref[...] = jnp.dot(h, w_ref[...],
                          preferred_element_type=jnp.float32) * dinv


def _mid(counts, acc, g1, b1, w2):
    return pl.pallas_call(
        _mid_body,
        grid=(NPAD // BM,),
        in_specs=[pl.BlockSpec((NC, BM, D), lambda i: (0, i, 0)),
                  pl.BlockSpec((NC, BM, D), lambda i: (0, i, 0)),
                  pl.BlockSpec((BM, D), lambda i: (i, 0)),
                  pl.BlockSpec((1, D), lambda i: (0, 0)),
                  pl.BlockSpec((D, D), lambda i: (0, 0))],
        out_specs=pl.BlockSpec((BM, D), lambda i: (i, 0)),
        out_shape=jax.ShapeDtypeStruct((NPAD, D), jnp.float32),
    )(counts, acc, g1, b1, w2)


def _final_body(cnt_ref, acc_ref, g2_ref, b_ref, o_ref):
    dinv = _dinv_of(cnt_ref[...])
    o_ref[...] = (acc_ref[0] + acc_ref[1] + g2_ref[...]) * dinv + b_ref[...]


def _final(counts, acc, g2, b2):
    return pl.pallas_call(
        _final_body,
        grid=(NPAD // BM,),
        in_specs=[pl.BlockSpec((NC, BM, D), lambda i: (0, i, 0)),
                  pl.BlockSpec((NC, BM, D), lambda i: (0, i, 0)),
                  pl.BlockSpec((BM, D), lambda i: (i, 0)),
                  pl.BlockSpec((1, D), lambda i: (0, 0))],
        out_specs=pl.BlockSpec((BM, D), lambda i: (i, 0)),
        out_shape=jax.ShapeDtypeStruct((NPAD, D), jnp.float32),
    )(counts, acc, g2, b2)


def kernel(x, edge_index, W1, b1, W2, b2):
    xpad = jnp.pad(x, ((0, NPAD - N), (0, 0)))
    pad = jnp.full((EPAD - E,), NPAD - 1, jnp.int32)
    srcp = jnp.concatenate([edge_index[0], pad]).reshape(NW, CH, CHUNK)
    dstp = jnp.concatenate([edge_index[1], pad]).reshape(NW, CH, CHUNK)

    counts = _deg_kernel(dstp)
    hraw1 = _mm(xpad, W1)
    g1 = _fin1(counts, hraw1)
    acc1 = _scatter_kernel(g1, srcp, dstp)
    g2 = _mid(counts, acc1, g1, b1.reshape(1, D), W2)
    acc2 = _scatter_kernel(g2, srcp, dstp)
    out = _final(counts, acc2, g2, b2.reshape(1, D))
    return out[:N]
